# baseline (speedup 1.0000x reference)
import functools

import jax
import jax.numpy as jnp
from jax import lax
from jax.experimental import pallas as pl
from jax.experimental.pallas import tpu as pltpu
from jax.experimental.pallas import tpu_sc as plsc

L_SEQ = 4096
D_EMB = 1024
_NC, _NS = 2, 16
_NW = _NC * _NS
_ROWS_PER_W = L_SEQ // _NW

_mesh = plsc.VectorSubcoreMesh(core_axis_name="c", subcore_axis_name="s")


@functools.partial(
    pl.kernel,
    out_type=jax.ShapeDtypeStruct((L_SEQ, D_EMB), jnp.float32),
    mesh=_mesh,
    scratch_types=[pltpu.SemaphoreType.DMA],
)
def _sc_copy(table_hbm, out_hbm, sem):
    wid = lax.axis_index("s") * _NC + lax.axis_index("c")
    base = wid * _ROWS_PER_W
    pltpu.async_copy(
        table_hbm.at[pl.ds(base, _ROWS_PER_W)],
        out_hbm.at[pl.ds(base, _ROWS_PER_W)],
        sem,
    ).wait()


def kernel(input_ids, embedding):
    del input_ids
    return _sc_copy(embedding)[None]


# speedup vs baseline: 16.7654x; 16.7654x over previous
import functools

import jax
import jax.numpy as jnp
from jax import lax
from jax.experimental import pallas as pl
from jax.experimental.pallas import tpu as pltpu
from jax.experimental.pallas import tpu_sc as plsc

L_SEQ = 4096
D_EMB = 1024
_NC, _NS = 2, 16
_NW = _NC * _NS
_ROWS_PER_W = L_SEQ // _NW

_CHUNK = 32
_NCHUNK = _ROWS_PER_W // _CHUNK

_mesh = plsc.VectorSubcoreMesh(core_axis_name="c", subcore_axis_name="s")


@functools.partial(
    pl.kernel,
    out_type=jax.ShapeDtypeStruct((L_SEQ, D_EMB), jnp.float32),
    mesh=_mesh,
    scratch_types=[
        pltpu.VMEM((_CHUNK, D_EMB), jnp.float32),
        pltpu.VMEM((_CHUNK, D_EMB), jnp.float32),
        pltpu.SemaphoreType.DMA,
        pltpu.SemaphoreType.DMA,
        pltpu.SemaphoreType.DMA,
        pltpu.SemaphoreType.DMA,
    ],
)
def _sc_copy(table_hbm, out_hbm, buf0, buf1, ls0, ls1, ss0, ss1):
    wid = lax.axis_index("s") * _NC + lax.axis_index("c")
    base = wid * _ROWS_PER_W
    bufs = (buf0, buf1)
    lsems = (ls0, ls1)
    ssems = (ss0, ss1)

    def load(i):
        return pltpu.make_async_copy(
            table_hbm.at[pl.ds(base + i * _CHUNK, _CHUNK)],
            bufs[i % 2], lsems[i % 2])

    def store(i):
        return pltpu.make_async_copy(
            bufs[i % 2],
            out_hbm.at[pl.ds(base + i * _CHUNK, _CHUNK)], ssems[i % 2])

    load(0).start()
    load(1).start()
    for i in range(_NCHUNK):
        load(i).wait()
        store(i).start()
        if i + 2 < _NCHUNK:
            store(i).wait()
            load(i + 2).start()
    store(_NCHUNK - 2).wait()
    store(_NCHUNK - 1).wait()


def kernel(input_ids, embedding):
    del input_ids
    return _sc_copy(embedding)[None]


# speedup vs baseline: 16.9679x; 1.0121x over previous
import functools

import jax
import jax.numpy as jnp
from jax import lax
from jax.experimental import pallas as pl
from jax.experimental.pallas import tpu as pltpu
from jax.experimental.pallas import tpu_sc as plsc

L_SEQ = 4096
D_EMB = 1024
_NC, _NS = 2, 16
_NW = _NC * _NS
_ROWS_PER_W = L_SEQ // _NW

_CHUNK = 16
_NBUF = 4
_NCHUNK = _ROWS_PER_W // _CHUNK

_mesh = plsc.VectorSubcoreMesh(core_axis_name="c", subcore_axis_name="s")


@functools.partial(
    pl.kernel,
    out_type=jax.ShapeDtypeStruct((L_SEQ, D_EMB), jnp.float32),
    mesh=_mesh,
    scratch_types=(
        [pltpu.VMEM((_CHUNK, D_EMB), jnp.float32)] * _NBUF
        + [pltpu.SemaphoreType.DMA] * (2 * _NBUF)
    ),
)
def _sc_copy(table_hbm, out_hbm, *scratch):
    bufs = scratch[:_NBUF]
    lsems = scratch[_NBUF:2 * _NBUF]
    ssems = scratch[2 * _NBUF:]
    wid = lax.axis_index("s") * _NC + lax.axis_index("c")
    base = wid * _ROWS_PER_W

    def load(i):
        return pltpu.make_async_copy(
            table_hbm.at[pl.ds(base + i * _CHUNK, _CHUNK)],
            bufs[i % _NBUF], lsems[i % _NBUF])

    def store(i):
        return pltpu.make_async_copy(
            bufs[i % _NBUF],
            out_hbm.at[pl.ds(base + i * _CHUNK, _CHUNK)], ssems[i % _NBUF])

    for i in range(_NBUF):
        load(i).start()
    for i in range(_NCHUNK):
        load(i).wait()
        store(i).start()
        if i + _NBUF < _NCHUNK:
            store(i).wait()
            load(i + _NBUF).start()
    for i in range(_NCHUNK - _NBUF, _NCHUNK):
        store(i).wait()


def kernel(input_ids, embedding):
    del input_ids
    return _sc_copy(embedding)[None]


# speedup vs baseline: 17.5733x; 1.0357x over previous
import functools

import jax
import jax.numpy as jnp
from jax import lax
from jax.experimental import pallas as pl
from jax.experimental.pallas import tpu as pltpu
from jax.experimental.pallas import tpu_sc as plsc

L_SEQ = 4096
D_EMB = 1024
_NC, _NS = 2, 16
_NW = _NC * _NS
_ROWS_PER_W = L_SEQ // _NW

_CHUNK = 16
_NBUF = 7
_NCHUNK = _ROWS_PER_W // _CHUNK

_mesh = plsc.VectorSubcoreMesh(core_axis_name="c", subcore_axis_name="s")


@functools.partial(
    pl.kernel,
    out_type=jax.ShapeDtypeStruct((L_SEQ, D_EMB), jnp.float32),
    mesh=_mesh,
    scratch_types=(
        [pltpu.VMEM((_CHUNK, D_EMB), jnp.float32)] * _NBUF
        + [pltpu.SemaphoreType.DMA] * (2 * _NBUF)
    ),
)
def _sc_copy(table_hbm, out_hbm, *scratch):
    bufs = scratch[:_NBUF]
    lsems = scratch[_NBUF:2 * _NBUF]
    ssems = scratch[2 * _NBUF:]
    wid = lax.axis_index("s") * _NC + lax.axis_index("c")
    base = wid * _ROWS_PER_W

    def load(i):
        return pltpu.make_async_copy(
            table_hbm.at[pl.ds(base + i * _CHUNK, _CHUNK)],
            bufs[i % _NBUF], lsems[i % _NBUF])

    def store(i):
        return pltpu.make_async_copy(
            bufs[i % _NBUF],
            out_hbm.at[pl.ds(base + i * _CHUNK, _CHUNK)], ssems[i % _NBUF])

    for i in range(_NBUF):
        load(i).start()
    for i in range(_NCHUNK):
        load(i).wait()
        store(i).start()
        if i + _NBUF < _NCHUNK:
            store(i).wait()
            load(i + _NBUF).start()
    for i in range(_NCHUNK - _NBUF, _NCHUNK):
        store(i).wait()


def kernel(input_ids, embedding):
    del input_ids
    return _sc_copy(embedding)[None]
